# baseline (speedup 1.0000x reference)
import functools

import jax
import jax.numpy as jnp
from jax import lax
from jax.experimental import pallas as pl
from jax.experimental.pallas import tpu as pltpu, tpu_sc as plsc

B = 16384
D = 64
L = 16
CH = 128
NROWS = 1000000
PAIRW = 2 * D


def _make_sc_kernel():
    info = plsc.get_sparse_core_info()
    nc, ns = info.num_cores, info.num_subcores
    nw = nc * ns
    b_per_w = B // nw
    n_ch = b_per_w // CH
    mesh = plsc.VectorSubcoreMesh(core_axis_name="c", subcore_axis_name="s")

    @functools.partial(
        pl.kernel,
        mesh=mesh,
        compiler_params=pltpu.CompilerParams(needs_layout_passes=False),
        out_type=jax.ShapeDtypeStruct((B, D), jnp.float32),
        scratch_types=[
            pltpu.VMEM((b_per_w,), jnp.int32),
            pltpu.VMEM((b_per_w,), jnp.int32),
            pltpu.VMEM((b_per_w,), jnp.int32),
            pltpu.VMEM((b_per_w,), jnp.int32),
            pltpu.VMEM((b_per_w,), jnp.int32),
            pltpu.VMEM((b_per_w,), jnp.int32),
            pltpu.VMEM((CH, PAIRW), jnp.float32),
            pltpu.VMEM((CH, PAIRW), jnp.float32),
            pltpu.VMEM((CH, PAIRW), jnp.float32),
            pltpu.VMEM((CH, D), jnp.float32),
            pltpu.SemaphoreType.DMA,
            pltpu.SemaphoreType.DMA,
            pltpu.SemaphoreType.DMA,
        ],
    )
    def k(pred_idx_hbm, head_idx_hbm, tail_idx_hbm, const_hbm, pred_hbm,
          out_hbm, ppair_v, hpair_v, tpair_v, phalf_v, hhalf_v, thalf_v,
          p_v, h_v, t_v, o_v, sem0, sem1, sem2):
        wid = lax.axis_index("s") * nc + lax.axis_index("c")
        base = wid * b_per_w
        pltpu.sync_copy(pred_idx_hbm.at[pl.ds(base, b_per_w)], ppair_v)
        pltpu.sync_copy(head_idx_hbm.at[pl.ds(base, b_per_w)], hpair_v)
        pltpu.sync_copy(tail_idx_hbm.at[pl.ds(base, b_per_w)], tpair_v)

        def split_body(i, _):
            sl = pl.ds(i * L, L)
            for pair_v, half_v in ((ppair_v, phalf_v), (hpair_v, hhalf_v),
                                   (tpair_v, thalf_v)):
                v = pair_v[sl]
                half_v[sl] = lax.bitwise_and(v, 1) * D
                pair_v[sl] = lax.shift_right_logical(v, 1)
            return 0

        lax.fori_loop(0, b_per_w // L, split_body, 0)

        lane = lax.iota(jnp.int32, L)

        def chunk_body(ch, _):
            off = pl.multiple_of(ch * CH, 8)
            cp0 = pltpu.async_copy(
                pred_hbm.at[ppair_v.at[pl.ds(off, CH)]], p_v, sem0)
            cp1 = pltpu.async_copy(
                const_hbm.at[hpair_v.at[pl.ds(off, CH)]], h_v, sem1)
            cp2 = pltpu.async_copy(
                const_hbm.at[tpair_v.at[pl.ds(off, CH)]], t_v, sem2)
            cp0.wait()
            cp1.wait()
            cp2.wait()

            for g in range(CH // L):
                slot = lane + g * L
                ph = phalf_v[pl.ds(off + g * L, L)]
                hh = hhalf_v[pl.ds(off + g * L, L)]
                th = thalf_v[pl.ds(off + g * L, L)]

                def col_body(c, _):
                    cc = jnp.zeros((L,), jnp.int32) + c
                    val = (plsc.load_gather(p_v, [slot, ph + cc])
                           + plsc.load_gather(h_v, [slot, hh + cc])
                           - plsc.load_gather(t_v, [slot, th + cc]))
                    plsc.store_scatter(o_v, [slot, cc], val)
                    return 0

                lax.fori_loop(0, D, col_body, 0)
            pltpu.sync_copy(o_v, out_hbm.at[pl.ds(base + off, CH)])
            return 0

        lax.fori_loop(0, n_ch, chunk_body, 0)

    return k


_sc_kernel = _make_sc_kernel()


@jax.jit
def kernel(sub_indices, constant_table, predicate_table):
    pred_idx = sub_indices[:, 0]
    head_idx = sub_indices[:, 1]
    tail_idx = sub_indices[:, 2]
    const_pairs = constant_table[:NROWS].reshape(NROWS // 2, PAIRW)
    pred_pairs = predicate_table[:NROWS].reshape(NROWS // 2, PAIRW)
    return _sc_kernel(pred_idx, head_idx, tail_idx, const_pairs, pred_pairs)


# speedup vs baseline: 1.0586x; 1.0586x over previous
import functools

import jax
import jax.numpy as jnp
from jax import lax
from jax.experimental import pallas as pl
from jax.experimental.pallas import tpu as pltpu, tpu_sc as plsc

B = 16384
D = 64
L = 16
CH = 128


def _make_sc_kernel():
    info = plsc.get_sparse_core_info()
    nc, ns = info.num_cores, info.num_subcores
    nw = nc * ns
    b_per_w = B // nw
    n_ch = b_per_w // CH
    mesh = plsc.VectorSubcoreMesh(core_axis_name="c", subcore_axis_name="s")

    @functools.partial(
        pl.kernel,
        mesh=mesh,
        compiler_params=pltpu.CompilerParams(use_tc_tiling_on_sc=False),
        out_type=jax.ShapeDtypeStruct((B * D,), jnp.float32),
        scratch_types=[
            pltpu.VMEM((b_per_w,), jnp.int32),
            pltpu.VMEM((b_per_w,), jnp.int32),
            pltpu.VMEM((b_per_w,), jnp.int32),
            pltpu.VMEM((CH, D), jnp.float32),
            pltpu.VMEM((CH, D), jnp.float32),
            pltpu.VMEM((CH, D), jnp.float32),
            pltpu.VMEM((b_per_w * D,), jnp.float32),
            pltpu.SemaphoreType.DMA,
            pltpu.SemaphoreType.DMA,
            pltpu.SemaphoreType.DMA,
        ],
    )
    def k(pred_idx_hbm, head_idx_hbm, tail_idx_hbm, const_hbm, pred_hbm,
          out_hbm, pidx_v, hidx_v, tidx_v, p_v, h_v, t_v, o_v,
          sem0, sem1, sem2):
        wid = lax.axis_index("s") * nc + lax.axis_index("c")
        base = wid * b_per_w
        pltpu.sync_copy(pred_idx_hbm.at[pl.ds(base, b_per_w)], pidx_v)
        pltpu.sync_copy(head_idx_hbm.at[pl.ds(base, b_per_w)], hidx_v)
        pltpu.sync_copy(tail_idx_hbm.at[pl.ds(base, b_per_w)], tidx_v)

        def chunk_body(ch, _):
            off = pl.multiple_of(ch * CH, 8)
            cp0 = pltpu.async_copy(
                pred_hbm.at[pidx_v.at[pl.ds(off, CH)]], p_v, sem0)
            cp1 = pltpu.async_copy(
                const_hbm.at[hidx_v.at[pl.ds(off, CH)]], h_v, sem1)
            cp2 = pltpu.async_copy(
                const_hbm.at[tidx_v.at[pl.ds(off, CH)]], t_v, sem2)
            cp0.wait()
            cp1.wait()
            cp2.wait()

            def combine_body(i, _):
                for j in range(D // L):
                    sl = pl.ds(j * L, L)
                    o_v[pl.ds((off + i) * D + j * L, L)] = (
                        p_v[i, sl] + h_v[i, sl] - t_v[i, sl])
                return 0

            lax.fori_loop(0, CH, combine_body, 0)
            return 0

        lax.fori_loop(0, n_ch, chunk_body, 0)
        pltpu.sync_copy(o_v, out_hbm.at[pl.ds(base * D, b_per_w * D)])

    return k


_sc_kernel = _make_sc_kernel()


@jax.jit
def kernel(sub_indices, constant_table, predicate_table):
    pred_idx = sub_indices[:, 0]
    head_idx = sub_indices[:, 1]
    tail_idx = sub_indices[:, 2]
    flat = _sc_kernel(pred_idx, head_idx, tail_idx, constant_table,
                      predicate_table)
    return flat.reshape(B, D)


# speedup vs baseline: 1.6639x; 1.5718x over previous
import functools

import jax
import jax.numpy as jnp
from jax import lax
from jax.experimental import pallas as pl
from jax.experimental.pallas import tpu as pltpu, tpu_sc as plsc

B = 16384
D = 64
L = 16
CH = 128


def _make_sc_kernel():
    info = plsc.get_sparse_core_info()
    nc, ns = info.num_cores, info.num_subcores
    nw = nc * ns
    b_per_w = B // nw
    n_ch = b_per_w // CH
    mesh = plsc.VectorSubcoreMesh(core_axis_name="c", subcore_axis_name="s")

    @functools.partial(
        pl.kernel,
        mesh=mesh,
        compiler_params=pltpu.CompilerParams(needs_layout_passes=False),
        out_type=jax.ShapeDtypeStruct((B, D), jnp.float32),
        scratch_types=[
            pltpu.VMEM((b_per_w,), jnp.int32),
            pltpu.VMEM((b_per_w,), jnp.int32),
            pltpu.VMEM((b_per_w,), jnp.int32),
            pltpu.VMEM((CH, D), jnp.float32),
            pltpu.VMEM((CH, D), jnp.float32),
            pltpu.VMEM((CH, D), jnp.float32),
            pltpu.SemaphoreType.DMA,
        ],
    )
    def k(pred_idx_hbm, head_idx_hbm, tail_idx_hbm, const_hbm, pred_hbm,
          out_hbm, pidx_v, hidx_v, tidx_v, p_v, h_v, t_v, sem):
        wid = lax.axis_index("s") * nc + lax.axis_index("c")
        base = wid * b_per_w
        pltpu.sync_copy(pred_idx_hbm.at[pl.ds(base, b_per_w)], pidx_v)
        pltpu.sync_copy(head_idx_hbm.at[pl.ds(base, b_per_w)], hidx_v)
        pltpu.sync_copy(tail_idx_hbm.at[pl.ds(base, b_per_w)], tidx_v)

        def chunk_body(ch, _):
            off = pl.multiple_of(ch * CH, 8)

            def issue_body(g, _):
                sl = pl.ds(off + g * L, L)
                for idx_v, tbl, dst in ((pidx_v, pred_hbm, p_v),
                                        (hidx_v, const_hbm, h_v),
                                        (tidx_v, const_hbm, t_v)):
                    vec = idx_v[sl]
                    for j in range(L):
                        r = jnp.squeeze(lax.slice(vec, (j,), (j + 1,)))
                        pltpu.async_copy(tbl.at[r], dst.at[g * L + j], sem)
                return 0

            lax.fori_loop(0, CH // L, issue_body, 0)

            def drain_body(i, _):
                pltpu.make_async_copy(pred_hbm.at[0], p_v.at[0], sem).wait()
                return 0

            lax.fori_loop(0, 3 * CH, drain_body, 0)

            def combine_body(i, _):
                for j in range(D // L):
                    sl = pl.ds(j * L, L)
                    p_v[i, sl] = p_v[i, sl] + h_v[i, sl] - t_v[i, sl]
                return 0

            lax.fori_loop(0, CH, combine_body, 0)
            pltpu.sync_copy(p_v, out_hbm.at[pl.ds(base + off, CH)])
            return 0

        lax.fori_loop(0, n_ch, chunk_body, 0)

    return k


_sc_kernel = _make_sc_kernel()


@jax.jit
def kernel(sub_indices, constant_table, predicate_table):
    pred_idx = sub_indices[:, 0]
    head_idx = sub_indices[:, 1]
    tail_idx = sub_indices[:, 2]
    return _sc_kernel(pred_idx, head_idx, tail_idx, constant_table,
                      predicate_table)
